# baseline (speedup 1.0000x reference)
import functools

import jax
import jax.numpy as jnp
from jax import lax
from jax.experimental import pallas as pl
from jax.experimental.pallas import tpu as pltpu
from jax.experimental.pallas import tpu_sc as plsc

_SCALE_H = 2.05
_SCALE_W = 3.15
_H = 512
_W = 512
_OH = int(round(_H * _SCALE_H))
_OW = int(round(_W * _SCALE_W))
_WPAD = 1616
_NC = 2
_NS = 16
_NW = _NC * _NS
_RCH = 33
_WIN = 20
_NFULL = _OW // 16


def _trace_indices():
    oy = jnp.arange(_OH)
    ox = jnp.arange(_OW)
    iy1 = jnp.floor(oy.astype(jnp.float32) / _SCALE_H).astype(jnp.int32)
    ix1 = jnp.floor(ox.astype(jnp.float32) / _SCALE_W).astype(jnp.int32)
    iy1 = jnp.clip(iy1, 0, _H - 1)
    ix1 = jnp.clip(ix1, 0, _W - 1)
    fy = (oy.astype(jnp.float32) + 0.5) / _SCALE_H - 0.5
    fx = (ox.astype(jnp.float32) + 0.5) / _SCALE_W - 0.5
    iy2 = jnp.clip(jnp.round(fy).astype(jnp.int32), 0, _H - 1)
    ix2 = jnp.clip(jnp.round(fx).astype(jnp.int32), 0, _W - 1)
    ix1p = jnp.pad(ix1, (0, _WPAD - _OW), mode="edge")
    ix2p = jnp.pad(ix2, (0, _WPAD - _OW), mode="edge")

    def chunks(iy):
        head = iy[: (_NW - 1) * _RCH].reshape(_NW - 1, _RCH)
        tail = iy[_OH - _RCH :][None]
        return jnp.concatenate([head, tail], axis=0)

    return ix1p, ix2p, chunks(iy1), chunks(iy2)


@functools.lru_cache(maxsize=1)
def _build():
    mesh = plsc.VectorSubcoreMesh(
        core_axis_name="c", subcore_axis_name="s", num_cores=_NC, num_subcores=_NS
    )

    @functools.partial(
        pl.kernel,
        out_type=(
            jax.ShapeDtypeStruct((_OH, _OW), jnp.float32),
            jax.ShapeDtypeStruct((_OH, _OW), jnp.float32),
        ),
        mesh=mesh,
        compiler_params=pltpu.CompilerParams(
            use_tc_tiling_on_sc=False, needs_layout_passes=False
        ),
        scratch_types=[
            pltpu.VMEM((_WIN, _W), jnp.float32),
            pltpu.VMEM((_WPAD,), jnp.int32),
            pltpu.VMEM((_WPAD,), jnp.int32),
            pltpu.VMEM((_RCH,), jnp.int32),
            pltpu.VMEM((_RCH,), jnp.int32),
            pltpu.VMEM((_RCH, _OW), jnp.float32),
            pltpu.VMEM((_RCH, _OW), jnp.float32),
            pltpu.SemaphoreType.DMA,
        ],
    )
    def resize(
        x_hbm, cx1_hbm, cx2_hbm, iy1_hbm, iy2_hbm, o1_hbm, o2_hbm,
        xwin, cx1v, cx2v, iy1v, iy2v, rows1, rows2, wsem,
    ):
        wid = lax.axis_index("s") * _NC + lax.axis_index("c")
        base = jnp.minimum(wid * _RCH, _OH - _RCH)
        sy0 = jnp.clip((20 * base) // 41 - 1, 0, _H - _WIN)
        pltpu.sync_copy(x_hbm.at[pl.ds(sy0, _WIN), :], xwin)
        pltpu.sync_copy(cx1_hbm, cx1v)
        pltpu.sync_copy(cx2_hbm, cx2v)
        pltpu.sync_copy(iy1_hbm.at[wid], iy1v)
        pltpu.sync_copy(iy2_hbm.at[wid], iy2v)
        sy0v = jnp.full((16,), sy0, jnp.int32)
        lanes = lax.iota(jnp.int32, 16)
        tailcol = _NFULL * 16 + lanes
        tailmask = tailcol < _OW

        def make_rows(iyv, cxv, rows):
            def jbody(j, carry):
                jv = jnp.full((16,), j, jnp.int32)
                wrel = jnp.clip(plsc.load_gather(iyv, [jv]) - sy0v, 0, _WIN - 1)

                def cbody(c, carry2):
                    c0 = c * 64
                    for k in range(4):
                        colv = cxv[pl.ds(c0 + k * 16, 16)]
                        rows[j, pl.ds(c0 + k * 16, 16)] = plsc.load_gather(
                            xwin, [wrel, colv]
                        )
                    return carry2

                lax.fori_loop(0, _NFULL // 4, cbody, 0)
                tailv = plsc.load_gather(xwin, [wrel, cxv[pl.ds(_NFULL * 16, 16)]])
                plsc.store_scatter(rows, [jv, tailcol], tailv, mask=tailmask)
                return carry

            lax.fori_loop(0, _RCH, jbody, 0)

        make_rows(iy1v, cx1v, rows1)
        write1 = pltpu.async_copy(rows1, o1_hbm.at[pl.ds(base, _RCH), :], wsem)
        make_rows(iy2v, cx2v, rows2)
        write2 = pltpu.async_copy(rows2, o2_hbm.at[pl.ds(base, _RCH), :], wsem)
        write1.wait()
        write2.wait()

    return resize


def kernel(x):
    x2d = x[0, 0]
    cx1, cx2, iy1, iy2 = _trace_indices()
    resize = _build()
    o1, o2 = resize(x2d, cx1, cx2, iy1, iy2)
    return (o1[None, None], o2[None, None])


# speedup vs baseline: 1.9982x; 1.9982x over previous
import functools

import jax
import jax.numpy as jnp
from jax import lax
from jax.experimental import pallas as pl
from jax.experimental.pallas import tpu as pltpu
from jax.experimental.pallas import tpu_sc as plsc

_SCALE_H = 2.05
_SCALE_W = 3.15
_H = 512
_W = 512
_OH = int(round(_H * _SCALE_H))
_OW = int(round(_W * _SCALE_W))
_WPAD = 1664
_NC = 2
_NS = 16
_NW = _NC * _NS
_NT = (_OH + 7) // 8
_WIN = 32
_NFULL = _OW // 16
_IYPAD = 1088


def _trace_indices():
    oy = jnp.arange(_OH)
    ox = jnp.arange(_OW)
    iy1 = jnp.floor(oy.astype(jnp.float32) / _SCALE_H).astype(jnp.int32)
    ix1 = jnp.floor(ox.astype(jnp.float32) / _SCALE_W).astype(jnp.int32)
    iy1 = jnp.clip(iy1, 0, _H - 1)
    ix1 = jnp.clip(ix1, 0, _W - 1)
    fy = (oy.astype(jnp.float32) + 0.5) / _SCALE_H - 0.5
    fx = (ox.astype(jnp.float32) + 0.5) / _SCALE_W - 0.5
    iy2 = jnp.clip(jnp.round(fy).astype(jnp.int32), 0, _H - 1)
    ix2 = jnp.clip(jnp.round(fx).astype(jnp.int32), 0, _W - 1)
    ix1p = jnp.pad(ix1, (0, _WPAD - _OW), mode="edge")
    ix2p = jnp.pad(ix2, (0, _WPAD - _OW), mode="edge")
    iy1p = jnp.pad(iy1, (8, _IYPAD - 8 - _OH), mode="edge")
    iy2p = jnp.pad(iy2, (8, _IYPAD - 8 - _OH), mode="edge")
    return ix1p, ix2p, iy1p, iy2p


@functools.lru_cache(maxsize=1)
def _build():
    mesh = plsc.VectorSubcoreMesh(
        core_axis_name="c", subcore_axis_name="s", num_cores=_NC, num_subcores=_NS
    )

    @functools.partial(
        pl.kernel,
        out_type=(
            jax.ShapeDtypeStruct((1, 1, _OH, _OW), jnp.float32),
            jax.ShapeDtypeStruct((1, 1, _OH, _OW), jnp.float32),
        ),
        mesh=mesh,
        compiler_params=pltpu.CompilerParams(
            use_tc_tiling_on_sc=True, needs_layout_passes=False
        ),
        scratch_types=[
            pltpu.VMEM((_WIN, _W), jnp.float32),
            pltpu.VMEM((_WIN * _W,), jnp.float32),
            pltpu.VMEM((_WPAD,), jnp.int32),
            pltpu.VMEM((_WPAD,), jnp.int32),
            pltpu.VMEM((56,), jnp.int32),
            pltpu.VMEM((56,), jnp.int32),
            pltpu.VMEM((8, _OW), jnp.float32),
            pltpu.VMEM((8, _OW), jnp.float32),
        ],
    )
    def resize(
        x_hbm, cx1_hbm, cx2_hbm, iy1_hbm, iy2_hbm, o1_hbm, o2_hbm,
        xw, lin, cx1v, cx2v, iy1v, iy2v, ot1, ot2,
    ):
        wid = lax.axis_index("s") * _NC + lax.axis_index("c")
        tstart = 4 * wid + jnp.minimum(wid, 4)
        rowstart = 8 * tstart
        win0 = pl.multiple_of(
            jnp.clip(((20 * rowstart) // 41 - 1) // 8 * 8, 0, _H - _WIN), 8
        )
        pltpu.sync_copy(x_hbm.at[pl.ds(win0, _WIN), :], xw)
        pltpu.sync_copy(cx1_hbm, cx1v)
        pltpu.sync_copy(cx2_hbm, cx2v)
        pltpu.sync_copy(iy1_hbm.at[pl.ds(rowstart, 56)], iy1v)
        pltpu.sync_copy(iy2_hbm.at[pl.ds(rowstart, 56)], iy2v)

        def rbody(r, carry):
            def cb(c, carry2):
                lin[pl.ds(r * _W + c * 16, 16)] = xw[r, pl.ds(c * 16, 16)]
                return carry2

            lax.fori_loop(0, _W // 16, cb, 0)
            return carry

        lax.fori_loop(0, _WIN, rbody, 0)

        win0v = jnp.full((16,), win0 * _W, jnp.int32)
        lanes = lax.iota(jnp.int32, 16)
        tailcol = _NFULL * 16 + lanes
        tailmask = tailcol < _OW

        def do_tile(k):
            t = tstart + k
            for jj0 in (0, 4):
                offs = []
                for jj in range(jj0, jj0 + 4):
                    jv = jnp.full((16,), 8 + 8 * k + jj, jnp.int32)
                    o1 = jnp.clip(
                        plsc.load_gather(iy1v, [jv]) * _W - win0v,
                        0, (_WIN - 1) * _W,
                    )
                    o2 = jnp.clip(
                        plsc.load_gather(iy2v, [jv]) * _W - win0v,
                        0, (_WIN - 1) * _W,
                    )
                    offs.append((o1, o2))

                def cbody(c, carry, offs=offs, jj0=jj0):
                    colv1 = cx1v[pl.ds(c * 16, 16)]
                    colv2 = cx2v[pl.ds(c * 16, 16)]
                    for i, (o1, o2) in enumerate(offs):
                        jj = jj0 + i
                        ot1[jj, pl.ds(c * 16, 16)] = plsc.load_gather(
                            lin, [o1 + colv1]
                        )
                        ot2[jj, pl.ds(c * 16, 16)] = plsc.load_gather(
                            lin, [o2 + colv2]
                        )
                    return carry

                lax.fori_loop(0, _NFULL, cbody, 0)
                tcol1 = cx1v[pl.ds(_NFULL * 16, 16)]
                tcol2 = cx2v[pl.ds(_NFULL * 16, 16)]
                for i, (o1, o2) in enumerate(offs):
                    jv = jnp.full((16,), jj0 + i, jnp.int32)
                    v1 = plsc.load_gather(lin, [o1 + tcol1])
                    v2 = plsc.load_gather(lin, [o2 + tcol2])
                    plsc.store_scatter(ot1, [jv, tailcol], v1, mask=tailmask)
                    plsc.store_scatter(ot2, [jv, tailcol], v2, mask=tailmask)

            @pl.when(t < _NT - 1)
            def _():
                pltpu.sync_copy(ot1, o1_hbm.at[0, 0, pl.ds(8 * t, 8), :])
                pltpu.sync_copy(ot2, o2_hbm.at[0, 0, pl.ds(8 * t, 8), :])

            @pl.when(t == _NT - 1)
            def _():
                pltpu.sync_copy(
                    ot1.at[pl.ds(0, _OH - 8 * (_NT - 1)), :],
                    o1_hbm.at[0, 0, pl.ds(8 * (_NT - 1), _OH - 8 * (_NT - 1)), :],
                )
                pltpu.sync_copy(
                    ot2.at[pl.ds(0, _OH - 8 * (_NT - 1)), :],
                    o2_hbm.at[0, 0, pl.ds(8 * (_NT - 1), _OH - 8 * (_NT - 1)), :],
                )

        for k in range(4):
            do_tile(k)

        @pl.when(wid < 4)
        def _():
            do_tile(4)

    return resize


def kernel(x):
    x2d = x[0, 0]
    cx1, cx2, iy1, iy2 = _trace_indices()
    resize = _build()
    o1, o2 = resize(x2d, cx1, cx2, iy1, iy2)
    return (o1, o2)


# speedup vs baseline: 3.4375x; 1.7203x over previous
import functools

import jax
import jax.numpy as jnp
from jax import lax
from jax.experimental import pallas as pl
from jax.experimental.pallas import tpu as pltpu
from jax.experimental.pallas import tpu_sc as plsc

_SCALE_H = 2.05
_SCALE_W = 3.15
_H = 512
_W = 512
_OH = int(round(_H * _SCALE_H))
_OW = int(round(_W * _SCALE_W))
_WPAD = 1664
_NC = 2
_NS = 16
_NW = _NC * _NS
_NT = (_OH + 7) // 8
_WIN = 32
_NFULL = _OW // 16
_IYPAD = 1088


def _trace_indices():
    oy = jnp.arange(_OH)
    ox = jnp.arange(_OW)
    iy1 = jnp.floor(oy.astype(jnp.float32) / _SCALE_H).astype(jnp.int32)
    ix1 = jnp.floor(ox.astype(jnp.float32) / _SCALE_W).astype(jnp.int32)
    iy1 = jnp.clip(iy1, 0, _H - 1)
    ix1 = jnp.clip(ix1, 0, _W - 1)
    fy = (oy.astype(jnp.float32) + 0.5) / _SCALE_H - 0.5
    fx = (ox.astype(jnp.float32) + 0.5) / _SCALE_W - 0.5
    iy2 = jnp.clip(jnp.round(fy).astype(jnp.int32), 0, _H - 1)
    ix2 = jnp.clip(jnp.round(fx).astype(jnp.int32), 0, _W - 1)
    ix1p = jnp.pad(ix1, (0, _WPAD - _OW), mode="edge")
    ix2p = jnp.pad(ix2, (0, _WPAD - _OW), mode="edge")
    iy1p = jnp.pad(iy1, (8, _IYPAD - 8 - _OH), mode="edge")
    iy2p = jnp.pad(iy2, (8, _IYPAD - 8 - _OH), mode="edge")
    return ix1p, ix2p, iy1p, iy2p


@functools.lru_cache(maxsize=1)
def _build():
    mesh = plsc.VectorSubcoreMesh(
        core_axis_name="c", subcore_axis_name="s", num_cores=_NC, num_subcores=_NS
    )

    @functools.partial(
        pl.kernel,
        out_type=(
            jax.ShapeDtypeStruct((1, 1, _OH, _OW), jnp.float32),
            jax.ShapeDtypeStruct((1, 1, _OH, _OW), jnp.float32),
        ),
        mesh=mesh,
        compiler_params=pltpu.CompilerParams(
            use_tc_tiling_on_sc=True, needs_layout_passes=False
        ),
        scratch_types=[
            pltpu.VMEM((_WIN, _W), jnp.float32),
            pltpu.VMEM((_WIN * _W,), jnp.float32),
            pltpu.VMEM((_WPAD,), jnp.int32),
            pltpu.VMEM((_WPAD,), jnp.int32),
            pltpu.VMEM((56,), jnp.int32),
            pltpu.VMEM((56,), jnp.int32),
            pltpu.VMEM((8, _OW), jnp.float32),
            pltpu.VMEM((8, _OW), jnp.float32),
            pltpu.VMEM((8, _OW), jnp.float32),
            pltpu.VMEM((8, _OW), jnp.float32),
            pltpu.VMEM((8, _OW), jnp.float32),
            pltpu.VMEM((8, _OW), jnp.float32),
            pltpu.SemaphoreType.DMA,
            pltpu.SemaphoreType.DMA,
        ],
    )
    def resize(
        x_hbm, cx1_hbm, cx2_hbm, iy1_hbm, iy2_hbm, o1_hbm, o2_hbm,
        xw, lin, cx1v, cx2v, iy1v, iy2v,
        ot1a, ot2a, ot1b, ot2b, ot1c, ot2c, insem, wsem,
    ):
        wid = lax.axis_index("s") * _NC + lax.axis_index("c")
        tstart = 4 * wid + jnp.minimum(wid, 4)
        rowstart = 8 * tstart
        win0 = pl.multiple_of(
            jnp.clip(((20 * rowstart) // 41 - 1) // 8 * 8, 0, _H - _WIN), 8
        )
        ins = [
            pltpu.async_copy(x_hbm.at[pl.ds(win0, _WIN), :], xw, insem),
            pltpu.async_copy(cx1_hbm, cx1v, insem),
            pltpu.async_copy(cx2_hbm, cx2v, insem),
            pltpu.async_copy(iy1_hbm.at[pl.ds(rowstart, 56)], iy1v, insem),
            pltpu.async_copy(iy2_hbm.at[pl.ds(rowstart, 56)], iy2v, insem),
        ]
        for cp in ins:
            cp.wait()

        def rbody(r):
            @plsc.parallel_loop(0, _W // 16, unroll=4)
            def cb(c):
                lin[pl.ds(r * _W + c * 16, 16)] = xw[r, pl.ds(c * 16, 16)]

        lax.fori_loop(0, _WIN, lambda r, carry: (rbody(r), carry)[1], 0)

        win0v = jnp.full((16,), win0 * _W, jnp.int32)
        lanes = lax.iota(jnp.int32, 16)
        tailcol = _NFULL * 16 + lanes
        tailmask = tailcol < _OW

        def do_tile(k, ot1, ot2, first_wait=None):
            t = tstart + k
            if first_wait is not None:
                for w in first_wait:
                    w.wait()
            for jj0 in (0, 4):
                offs = []
                for jj in range(jj0, jj0 + 4):
                    jv = jnp.full((16,), 8 + 8 * k + jj, jnp.int32)
                    o1 = jnp.clip(
                        plsc.load_gather(iy1v, [jv]) * _W - win0v,
                        0, (_WIN - 1) * _W,
                    )
                    o2 = jnp.clip(
                        plsc.load_gather(iy2v, [jv]) * _W - win0v,
                        0, (_WIN - 1) * _W,
                    )
                    offs.append((o1, o2))

                @plsc.parallel_loop(0, _NFULL, unroll=2)
                def cbody(c, offs=offs, jj0=jj0, ot1=ot1, ot2=ot2):
                    colv1 = cx1v[pl.ds(c * 16, 16)]
                    colv2 = cx2v[pl.ds(c * 16, 16)]
                    for i, (o1, o2) in enumerate(offs):
                        jj = jj0 + i
                        ot1[jj, pl.ds(c * 16, 16)] = plsc.load_gather(
                            lin, [o1 + colv1]
                        )
                        ot2[jj, pl.ds(c * 16, 16)] = plsc.load_gather(
                            lin, [o2 + colv2]
                        )
                tcol1 = cx1v[pl.ds(_NFULL * 16, 16)]
                tcol2 = cx2v[pl.ds(_NFULL * 16, 16)]
                for i, (o1, o2) in enumerate(offs):
                    jv = jnp.full((16,), jj0 + i, jnp.int32)
                    v1 = plsc.load_gather(lin, [o1 + tcol1])
                    v2 = plsc.load_gather(lin, [o2 + tcol2])
                    plsc.store_scatter(ot1, [jv, tailcol], v1, mask=tailmask)
                    plsc.store_scatter(ot2, [jv, tailcol], v2, mask=tailmask)

            if k != 3:
                w1 = pltpu.async_copy(ot1, o1_hbm.at[0, 0, pl.ds(8 * t, 8), :], wsem)
                w2 = pltpu.async_copy(ot2, o2_hbm.at[0, 0, pl.ds(8 * t, 8), :], wsem)
                return (w1, w2)

            @pl.when(t < _NT - 1)
            def _():
                w1 = pltpu.async_copy(ot1, o1_hbm.at[0, 0, pl.ds(8 * t, 8), :], wsem)
                w2 = pltpu.async_copy(ot2, o2_hbm.at[0, 0, pl.ds(8 * t, 8), :], wsem)
                w1.wait()
                w2.wait()

            @pl.when(t == _NT - 1)
            def _():
                w1 = pltpu.async_copy(
                    ot1.at[pl.ds(0, _OH - 8 * (_NT - 1)), :],
                    o1_hbm.at[0, 0, pl.ds(8 * (_NT - 1), _OH - 8 * (_NT - 1)), :],
                    wsem,
                )
                w2 = pltpu.async_copy(
                    ot2.at[pl.ds(0, _OH - 8 * (_NT - 1)), :],
                    o2_hbm.at[0, 0, pl.ds(8 * (_NT - 1), _OH - 8 * (_NT - 1)), :],
                    wsem,
                )
                w1.wait()
                w2.wait()
            return None

        w0 = do_tile(0, ot1a, ot2a)
        w1 = do_tile(1, ot1b, ot2b)
        w2 = do_tile(2, ot1a, ot2a, first_wait=w0)
        do_tile(3, ot1b, ot2b, first_wait=w1)

        @pl.when(wid < 4)
        def _():
            w4 = do_tile(4, ot1c, ot2c)
            for w in w4:
                w.wait()

        for w in w2:
            w.wait()

    return resize


def kernel(x):
    x2d = x[0, 0]
    with jax.ensure_compile_time_eval():
        cx1, cx2, iy1, iy2 = _trace_indices()
    resize = _build()
    o1, o2 = resize(x2d, cx1, cx2, iy1, iy2)
    return (o1, o2)
